# Initial kernel scaffold; baseline (speedup 1.0000x reference)
import numpy as np
import jax
import jax.numpy as jnp
from jax.experimental import pallas as pl

N = 1000
NP = 1024
C = 81
LANES = 128
SCORE_THRESH = 0.05
NMS_THRESH = 0.5
DET = 100
DET_PAD = 104
WX, WY, WW, WH = 10.0, 10.0, 5.0, 5.0
BBOX_XFORM_CLIP = float(np.log(1000.0 / 16.0))
IMG_W, IMG_H = 1333.0, 800.0
NEG = -1e30
BIG = np.int32(2**30)

_INTERPRET = False


def _body(logits_ref, dx_ref, dy_ref, dw_ref, dh_ref, props_ref,
          det_ref, lab_ref):
    rows = jax.lax.broadcasted_iota(jnp.int32, (NP, 1), 0)
    lanes = jax.lax.broadcasted_iota(jnp.int32, (1, LANES), 1)
    row_idx = jax.lax.broadcasted_iota(jnp.int32, (NP, LANES), 0)

    lg = logits_ref[...]
    mx = jnp.max(lg, axis=1, keepdims=True)
    ex = jnp.exp(lg - mx)
    probs = ex / jnp.sum(ex, axis=1, keepdims=True)

    p = props_ref[...]
    px1, py1, px2, py2 = p[:, 0:1], p[:, 1:2], p[:, 2:3], p[:, 3:4]
    widths = px2 - px1 + 1.0
    heights = py2 - py1 + 1.0
    ctr_x = px1 + 0.5 * widths
    ctr_y = py1 + 0.5 * heights
    dx = dx_ref[...] / WX
    dy = dy_ref[...] / WY
    dw = jnp.minimum(dw_ref[...] / WW, BBOX_XFORM_CLIP)
    dh = jnp.minimum(dh_ref[...] / WH, BBOX_XFORM_CLIP)
    pcx = dx * widths + ctr_x
    pcy = dy * heights + ctr_y
    pw = jnp.exp(dw) * widths
    ph = jnp.exp(dh) * heights
    x1 = jnp.clip(pcx - 0.5 * pw, 0.0, IMG_W - 1.0)
    y1 = jnp.clip(pcy - 0.5 * ph, 0.0, IMG_H - 1.0)
    x2 = jnp.clip(pcx + 0.5 * pw - 1.0, 0.0, IMG_W - 1.0)
    y2 = jnp.clip(pcy + 0.5 * ph - 1.0, 0.0, IMG_H - 1.0)

    lane_ok = (lanes >= 1) & (lanes <= C - 1)
    real = lane_ok & (rows < N)
    valid = real & (probs > SCORE_THRESH)
    key = jnp.where(valid, probs, NEG)
    idx = row_idx

    def shuffle(x, j, is_lower):
        return jnp.where(is_lower, jnp.roll(x, -j, axis=0), jnp.roll(x, j, axis=0))

    k = 2
    while k <= NP:
        j = k // 2
        while j >= 1:
            is_lower = (rows & j) == 0
            desc = (rows & k) == 0
            want_big = desc == is_lower
            ky = shuffle(key, j, is_lower)
            iy = shuffle(idx, j, is_lower)
            x1y = shuffle(x1, j, is_lower)
            y1y = shuffle(y1, j, is_lower)
            x2y = shuffle(x2, j, is_lower)
            y2y = shuffle(y2, j, is_lower)
            x_is_big = (key > ky) | ((key == ky) & (idx < iy))
            take = want_big ^ x_is_big
            key = jnp.where(take, ky, key)
            idx = jnp.where(take, iy, idx)
            x1 = jnp.where(take, x1y, x1)
            y1 = jnp.where(take, y1y, y1)
            x2 = jnp.where(take, x2y, x2)
            y2 = jnp.where(take, y2y, y2)
            j //= 2
        k *= 2

    area = (x2 - x1 + 1.0) * (y2 - y1 + 1.0)
    validk = key > SCORE_THRESH
    supp0 = jnp.logical_not(validk)
    counts = jnp.sum(validk.astype(jnp.int32), axis=0, keepdims=True)
    kmax = jnp.max(counts)

    def nms_body(i, supp):
        xi1 = jax.lax.dynamic_slice_in_dim(x1, i, 1, 0)
        yi1 = jax.lax.dynamic_slice_in_dim(y1, i, 1, 0)
        xi2 = jax.lax.dynamic_slice_in_dim(x2, i, 1, 0)
        yi2 = jax.lax.dynamic_slice_in_dim(y2, i, 1, 0)
        ai = jax.lax.dynamic_slice_in_dim(area, i, 1, 0)
        alive = jnp.logical_not(jax.lax.dynamic_slice_in_dim(supp, i, 1, 0))
        xx1 = jnp.maximum(x1, xi1)
        yy1 = jnp.maximum(y1, yi1)
        xx2 = jnp.minimum(x2, xi2)
        yy2 = jnp.minimum(y2, yi2)
        w = jnp.maximum(xx2 - xx1 + 1.0, 0.0)
        h = jnp.maximum(yy2 - yy1 + 1.0, 0.0)
        inter = w * h
        iou = inter / (area + ai - inter)
        newsup = alive & (iou > NMS_THRESH) & (row_idx > i)
        return supp | newsup

    supp = jax.lax.fori_loop(0, kmax, nms_body, supp0)

    keep = jnp.logical_not(supp)
    isreal = lane_ok & (idx < N)
    val = jnp.where(keep, key, jnp.where(isreal, -1.0, -2.0))
    fidx = jnp.where(isreal, (lanes - 1) * N + idx, BIG)
    det_ref[...] = jnp.zeros((DET_PAD, LANES), jnp.float32)

    def sel_body(t, carry):
        v, labv = carry
        m = jnp.max(v)
        cand = v == m
        wf = jnp.min(jnp.where(cand, fidx, BIG))
        winner = cand & (fidx == wf)
        bx1 = jnp.sum(jnp.where(winner, x1, 0.0))
        by1 = jnp.sum(jnp.where(winner, y1, 0.0))
        bx2 = jnp.sum(jnp.where(winner, x2, 0.0))
        by2 = jnp.sum(jnp.where(winner, y2, 0.0))
        label = wf // N + 1
        rowv = jnp.where(lanes == 0, bx1,
               jnp.where(lanes == 1, by1,
               jnp.where(lanes == 2, bx2,
               jnp.where(lanes == 3, by2,
               jnp.where(lanes == 4, m, 0.0)))))
        det_ref[pl.ds(t, 1), :] = rowv
        labv = jnp.where(lanes == t, label, labv)
        v = jnp.where(winner, -3.0, v)
        return v, labv

    _, labv = jax.lax.fori_loop(
        0, DET, sel_body, (val, jnp.zeros((1, LANES), jnp.int32)))
    lab_ref[...] = jnp.broadcast_to(labv, (8, LANES))


def kernel(class_logits, box_regression, proposals):
    lg = jnp.pad(class_logits, ((0, NP - N), (0, 0)))
    lg = jnp.pad(lg, ((0, 0), (0, LANES - C)), constant_values=NEG)
    r4 = box_regression.reshape(N, C, 4)

    def padc(a):
        return jnp.pad(a, ((0, NP - N), (0, LANES - C)))

    dxi = padc(r4[:, :, 0])
    dyi = padc(r4[:, :, 1])
    dwi = padc(r4[:, :, 2])
    dhi = padc(r4[:, :, 3])
    props = jnp.pad(proposals, ((0, NP - N), (0, 0)))
    det_full, lab_full = pl.pallas_call(
        _body,
        out_shape=[jax.ShapeDtypeStruct((DET_PAD, LANES), jnp.float32),
                   jax.ShapeDtypeStruct((8, LANES), jnp.int32)],
        interpret=_INTERPRET,
    )(lg, dxi, dyi, dwi, dhi, props)
    det = det_full[:DET, :5]
    labels = lab_full[0, :DET]
    return det, labels


# speedup vs baseline: 71.8159x; 71.8159x over previous
import numpy as np
import jax
import jax.numpy as jnp
from jax.experimental import pallas as pl
from jax.experimental.pallas import tpu as pltpu

N = 1000
NP = 1024
C = 81
LANES = 128
SCORE_THRESH = 0.05
NMS_THRESH = 0.5
DET = 100
DET_PAD = 104
WX, WY, WW, WH = 10.0, 10.0, 5.0, 5.0
BBOX_XFORM_CLIP = float(np.log(1000.0 / 16.0))
IMG_W, IMG_H = 1333.0, 800.0
NEG = -1e30
BIG = np.int32(2**30)

_INTERPRET = False


def _body(logits_ref, dx_ref, dy_ref, dw_ref, dh_ref, props_ref,
          det_ref, lab_ref, sx1, sy1, sx2, sy2, sar, ssup):
    rows = jax.lax.broadcasted_iota(jnp.int32, (NP, 1), 0)
    lanes = jax.lax.broadcasted_iota(jnp.int32, (1, LANES), 1)
    row_idx = jax.lax.broadcasted_iota(jnp.int32, (NP, LANES), 0)

    lg = logits_ref[...]
    mx = jnp.max(lg, axis=1, keepdims=True)
    ex = jnp.exp(lg - mx)
    probs = ex / jnp.sum(ex, axis=1, keepdims=True)

    p = props_ref[...]
    px1, py1, px2, py2 = p[:, 0:1], p[:, 1:2], p[:, 2:3], p[:, 3:4]
    widths = px2 - px1 + 1.0
    heights = py2 - py1 + 1.0
    ctr_x = px1 + 0.5 * widths
    ctr_y = py1 + 0.5 * heights
    dx = dx_ref[...] / WX
    dy = dy_ref[...] / WY
    dw = jnp.minimum(dw_ref[...] / WW, BBOX_XFORM_CLIP)
    dh = jnp.minimum(dh_ref[...] / WH, BBOX_XFORM_CLIP)
    pcx = dx * widths + ctr_x
    pcy = dy * heights + ctr_y
    pw = jnp.exp(dw) * widths
    ph = jnp.exp(dh) * heights
    x1 = jnp.clip(pcx - 0.5 * pw, 0.0, IMG_W - 1.0)
    y1 = jnp.clip(pcy - 0.5 * ph, 0.0, IMG_H - 1.0)
    x2 = jnp.clip(pcx + 0.5 * pw - 1.0, 0.0, IMG_W - 1.0)
    y2 = jnp.clip(pcy + 0.5 * ph - 1.0, 0.0, IMG_H - 1.0)

    lane_ok = (lanes >= 1) & (lanes <= C - 1)
    real = lane_ok & (rows < N)
    valid = real & (probs > SCORE_THRESH)
    key = jnp.where(valid, probs, NEG)
    idx = row_idx

    def stage(carry, j, k):
        key, idx, x1, y1, x2, y2 = carry
        is_lower = (rows & j) == 0
        desc = (rows & k) == 0
        want_big = desc == is_lower

        def shuf(x):
            up = pltpu.roll(x, NP - j, 0)
            dn = pltpu.roll(x, j, 0)
            return jnp.where(is_lower, up, dn)

        ky = shuf(key)
        iy = shuf(idx)
        x1y = shuf(x1)
        y1y = shuf(y1)
        x2y = shuf(x2)
        y2y = shuf(y2)
        x_is_big = (key > ky) | ((key == ky) & (idx < iy))
        take = want_big ^ x_is_big
        return (jnp.where(take, ky, key), jnp.where(take, iy, idx),
                jnp.where(take, x1y, x1), jnp.where(take, y1y, y1),
                jnp.where(take, x2y, x2), jnp.where(take, y2y, y2))

    def level_body(l, carry):
        k = jnp.left_shift(jnp.int32(1), l)

        def sub_body(t, c):
            j = jnp.right_shift(k, t + 1)
            return stage(c, j, k)

        return jax.lax.fori_loop(0, l, sub_body, carry)

    key, idx, x1, y1, x2, y2 = jax.lax.fori_loop(
        1, 11, level_body, (key, idx, x1, y1, x2, y2))

    area = (x2 - x1 + 1.0) * (y2 - y1 + 1.0)
    validk = key > SCORE_THRESH
    counts = jnp.sum(validk.astype(jnp.int32), axis=0, keepdims=True)
    kmax = jnp.max(counts)

    sx1[...] = x1
    sy1[...] = y1
    sx2[...] = x2
    sy2[...] = y2
    sar[...] = area
    ssup[...] = jnp.logical_not(validk).astype(jnp.int32)

    def nms_body(i, carry):
        xi1 = sx1[pl.ds(i, 1), :]
        yi1 = sy1[pl.ds(i, 1), :]
        xi2 = sx2[pl.ds(i, 1), :]
        yi2 = sy2[pl.ds(i, 1), :]
        ai = sar[pl.ds(i, 1), :]
        alive = ssup[pl.ds(i, 1), :] == 0
        xx1 = jnp.maximum(x1, xi1)
        yy1 = jnp.maximum(y1, yi1)
        xx2 = jnp.minimum(x2, xi2)
        yy2 = jnp.minimum(y2, yi2)
        w = jnp.maximum(xx2 - xx1 + 1.0, 0.0)
        h = jnp.maximum(yy2 - yy1 + 1.0, 0.0)
        inter = w * h
        iou = inter / (area + ai - inter)
        newsup = alive & (iou > NMS_THRESH) & (row_idx > i)
        ssup[...] = ssup[...] | newsup.astype(jnp.int32)
        return carry

    jax.lax.fori_loop(0, kmax, nms_body, 0)

    keep = ssup[...] == 0
    isreal = lane_ok & (idx < N)
    val = jnp.where(keep, key, jnp.where(isreal, -1.0, -2.0))
    fidx = jnp.where(isreal, (lanes - 1) * N + idx, BIG)
    det_ref[...] = jnp.zeros((DET_PAD, LANES), jnp.float32)

    def sel_body(t, carry):
        v, labv = carry
        m = jnp.max(v)
        cand = v == m
        wf = jnp.min(jnp.where(cand, fidx, BIG))
        winner = cand & (fidx == wf)
        bx1 = jnp.sum(jnp.where(winner, x1, 0.0))
        by1 = jnp.sum(jnp.where(winner, y1, 0.0))
        bx2 = jnp.sum(jnp.where(winner, x2, 0.0))
        by2 = jnp.sum(jnp.where(winner, y2, 0.0))
        label = wf // N + 1
        rowv = jnp.where(lanes == 0, bx1,
               jnp.where(lanes == 1, by1,
               jnp.where(lanes == 2, bx2,
               jnp.where(lanes == 3, by2,
               jnp.where(lanes == 4, m, 0.0)))))
        det_ref[pl.ds(t, 1), :] = rowv
        labv = jnp.where(lanes == t, label, labv)
        v = jnp.where(winner, -3.0, v)
        return v, labv

    _, labv = jax.lax.fori_loop(
        0, DET, sel_body, (val, jnp.zeros((1, LANES), jnp.int32)))
    lab_ref[...] = jnp.broadcast_to(labv, (8, LANES))


def kernel(class_logits, box_regression, proposals):
    lg = jnp.pad(class_logits, ((0, NP - N), (0, 0)))
    lg = jnp.pad(lg, ((0, 0), (0, LANES - C)), constant_values=NEG)
    r4 = box_regression.reshape(N, C, 4)

    def padc(a):
        return jnp.pad(a, ((0, NP - N), (0, LANES - C)))

    dxi = padc(r4[:, :, 0])
    dyi = padc(r4[:, :, 1])
    dwi = padc(r4[:, :, 2])
    dhi = padc(r4[:, :, 3])
    props = jnp.pad(proposals, ((0, NP - N), (0, 0)))
    det_full, lab_full = pl.pallas_call(
        _body,
        out_shape=[jax.ShapeDtypeStruct((DET_PAD, LANES), jnp.float32),
                   jax.ShapeDtypeStruct((8, LANES), jnp.int32)],
        scratch_shapes=[pltpu.VMEM((NP, LANES), jnp.float32)] * 5
                       + [pltpu.VMEM((NP, LANES), jnp.int32)],
        interpret=_INTERPRET,
    )(lg, dxi, dyi, dwi, dhi, props)
    det = det_full[:DET, :5]
    labels = lab_full[0, :DET]
    return det, labels


# speedup vs baseline: 182.0704x; 2.5352x over previous
import numpy as np
import jax
import jax.numpy as jnp
from jax.experimental import pallas as pl

N = 1000
NP = 1024
C = 81
LANES = 128
SCORE_THRESH = 0.05
NMS_THRESH = 0.5
DET = 100
DET_PAD = 104
WX, WY, WW, WH = 10.0, 10.0, 5.0, 5.0
BBOX_XFORM_CLIP = float(np.log(1000.0 / 16.0))
IMG_W, IMG_H = 1333.0, 800.0
NEG = -1e30
BIG = np.int32(2**30)

_INTERPRET = False


def _body(logits_ref, dx_ref, dy_ref, dw_ref, dh_ref, props_ref,
          det_ref, lab_ref):
    rows = jax.lax.broadcasted_iota(jnp.int32, (NP, 1), 0)
    lanes = jax.lax.broadcasted_iota(jnp.int32, (1, LANES), 1)
    row_idx = jax.lax.broadcasted_iota(jnp.int32, (NP, LANES), 0)

    lg = logits_ref[...]
    mx = jnp.max(lg, axis=1, keepdims=True)
    ex = jnp.exp(lg - mx)
    probs = ex / jnp.sum(ex, axis=1, keepdims=True)

    p = props_ref[...]
    px1, py1, px2, py2 = p[:, 0:1], p[:, 1:2], p[:, 2:3], p[:, 3:4]
    widths = px2 - px1 + 1.0
    heights = py2 - py1 + 1.0
    ctr_x = px1 + 0.5 * widths
    ctr_y = py1 + 0.5 * heights
    dx = dx_ref[...] / WX
    dy = dy_ref[...] / WY
    dw = jnp.minimum(dw_ref[...] / WW, BBOX_XFORM_CLIP)
    dh = jnp.minimum(dh_ref[...] / WH, BBOX_XFORM_CLIP)
    pcx = dx * widths + ctr_x
    pcy = dy * heights + ctr_y
    pw = jnp.exp(dw) * widths
    ph = jnp.exp(dh) * heights
    x1 = jnp.clip(pcx - 0.5 * pw, 0.0, IMG_W - 1.0)
    y1 = jnp.clip(pcy - 0.5 * ph, 0.0, IMG_H - 1.0)
    x2 = jnp.clip(pcx + 0.5 * pw - 1.0, 0.0, IMG_W - 1.0)
    y2 = jnp.clip(pcy + 0.5 * ph - 1.0, 0.0, IMG_H - 1.0)
    area = (x2 - x1 + 1.0) * (y2 - y1 + 1.0)

    lane_ok = (lanes >= 1) & (lanes <= C - 1)
    real = lane_ok & (rows < N)
    valid = real & (probs > SCORE_THRESH)

    work0 = jnp.where(valid, probs, -2.0)
    val0 = jnp.where(real, -1.0, -2.0)

    def nms_cond(carry):
        t, work, _ = carry
        return (t < DET) & (jnp.max(work) > SCORE_THRESH)

    def nms_body(carry):
        t, work, val = carry
        m = jnp.max(work, axis=0, keepdims=True)
        lane_active = m > SCORE_THRESH
        cand = work == m
        first = jnp.min(jnp.where(cand, row_idx, BIG), axis=0, keepdims=True)
        b = cand & (row_idx == first) & lane_active
        xi1 = jnp.sum(jnp.where(b, x1, 0.0), axis=0, keepdims=True)
        yi1 = jnp.sum(jnp.where(b, y1, 0.0), axis=0, keepdims=True)
        xi2 = jnp.sum(jnp.where(b, x2, 0.0), axis=0, keepdims=True)
        yi2 = jnp.sum(jnp.where(b, y2, 0.0), axis=0, keepdims=True)
        ai = jnp.sum(jnp.where(b, area, 0.0), axis=0, keepdims=True)
        xx1 = jnp.maximum(x1, xi1)
        yy1 = jnp.maximum(y1, yi1)
        xx2 = jnp.minimum(x2, xi2)
        yy2 = jnp.minimum(y2, yi2)
        w = jnp.maximum(xx2 - xx1 + 1.0, 0.0)
        h = jnp.maximum(yy2 - yy1 + 1.0, 0.0)
        inter = w * h
        iou = inter / (area + ai - inter)
        val = jnp.where(b, m, val)
        work = jnp.where(lane_active & (iou > NMS_THRESH), -2.0, work)
        return t + 1, work, val

    _, _, val = jax.lax.while_loop(nms_cond, nms_body, (0, work0, val0))

    fidx = jnp.where(real, (lanes - 1) * N + row_idx, BIG)
    det_ref[...] = jnp.zeros((DET_PAD, LANES), jnp.float32)

    def sel_body(t, carry):
        v, labv = carry
        m = jnp.max(v)
        cand = v == m
        wf = jnp.min(jnp.where(cand, fidx, BIG))
        winner = cand & (fidx == wf)
        bx1 = jnp.sum(jnp.where(winner, x1, 0.0))
        by1 = jnp.sum(jnp.where(winner, y1, 0.0))
        bx2 = jnp.sum(jnp.where(winner, x2, 0.0))
        by2 = jnp.sum(jnp.where(winner, y2, 0.0))
        label = wf // N + 1
        rowv = jnp.where(lanes == 0, bx1,
               jnp.where(lanes == 1, by1,
               jnp.where(lanes == 2, bx2,
               jnp.where(lanes == 3, by2,
               jnp.where(lanes == 4, m, 0.0)))))
        det_ref[pl.ds(t, 1), :] = rowv
        labv = jnp.where(lanes == t, label, labv)
        v = jnp.where(winner, -3.0, v)
        return v, labv

    _, labv = jax.lax.fori_loop(
        0, DET, sel_body, (val, jnp.zeros((1, LANES), jnp.int32)))
    lab_ref[...] = jnp.broadcast_to(labv, (8, LANES))


def kernel(class_logits, box_regression, proposals):
    lg = jnp.pad(class_logits, ((0, NP - N), (0, 0)))
    lg = jnp.pad(lg, ((0, 0), (0, LANES - C)), constant_values=NEG)
    r4 = box_regression.reshape(N, C, 4)

    def padc(a):
        return jnp.pad(a, ((0, NP - N), (0, LANES - C)))

    dxi = padc(r4[:, :, 0])
    dyi = padc(r4[:, :, 1])
    dwi = padc(r4[:, :, 2])
    dhi = padc(r4[:, :, 3])
    props = jnp.pad(proposals, ((0, NP - N), (0, 0)))
    det_full, lab_full = pl.pallas_call(
        _body,
        out_shape=[jax.ShapeDtypeStruct((DET_PAD, LANES), jnp.float32),
                   jax.ShapeDtypeStruct((8, LANES), jnp.int32)],
        interpret=_INTERPRET,
    )(lg, dxi, dyi, dwi, dhi, props)
    det = det_full[:DET, :5]
    labels = lab_full[0, :DET]
    return det, labels


# speedup vs baseline: 198.5948x; 1.0908x over previous
import numpy as np
import jax
import jax.numpy as jnp
from jax.experimental import pallas as pl
from jax.experimental.pallas import tpu as pltpu

N = 1000
NP = 1024
C = 81
LANES = 128
SCORE_THRESH = 0.05
NMS_THRESH = 0.5
DET = 100
DET_PAD = 104
WX, WY, WW, WH = 10.0, 10.0, 5.0, 5.0
BBOX_XFORM_CLIP = float(np.log(1000.0 / 16.0))
IMG_W, IMG_H = 1333.0, 800.0
NEG = -1e30
BIG = np.int32(2**30)

_INTERPRET = False


def _body(logits_ref, dx_ref, dy_ref, dw_ref, dh_ref, props_ref,
          det_ref, lab_ref, sx1, sy1, sx2, sy2):
    rows = jax.lax.broadcasted_iota(jnp.int32, (NP, 1), 0)
    lanes = jax.lax.broadcasted_iota(jnp.int32, (1, LANES), 1)
    row_idx = jax.lax.broadcasted_iota(jnp.int32, (NP, LANES), 0)

    lg = logits_ref[...]
    mx = jnp.max(lg, axis=1, keepdims=True)
    ex = jnp.exp(lg - mx)
    probs = ex / jnp.sum(ex, axis=1, keepdims=True)

    p = props_ref[...]
    px1, py1, px2, py2 = p[:, 0:1], p[:, 1:2], p[:, 2:3], p[:, 3:4]
    widths = px2 - px1 + 1.0
    heights = py2 - py1 + 1.0
    ctr_x = px1 + 0.5 * widths
    ctr_y = py1 + 0.5 * heights
    dx = dx_ref[...] / WX
    dy = dy_ref[...] / WY
    dw = jnp.minimum(dw_ref[...] / WW, BBOX_XFORM_CLIP)
    dh = jnp.minimum(dh_ref[...] / WH, BBOX_XFORM_CLIP)
    pcx = dx * widths + ctr_x
    pcy = dy * heights + ctr_y
    pw = jnp.exp(dw) * widths
    ph = jnp.exp(dh) * heights
    x1 = jnp.clip(pcx - 0.5 * pw, 0.0, IMG_W - 1.0)
    y1 = jnp.clip(pcy - 0.5 * ph, 0.0, IMG_H - 1.0)
    x2 = jnp.clip(pcx + 0.5 * pw - 1.0, 0.0, IMG_W - 1.0)
    y2 = jnp.clip(pcy + 0.5 * ph - 1.0, 0.0, IMG_H - 1.0)
    area = (x2 - x1 + 1.0) * (y2 - y1 + 1.0)

    lane_ok = (lanes >= 1) & (lanes <= C - 1)
    real = lane_ok & (rows < N)
    valid = real & (probs > SCORE_THRESH)

    work0 = jnp.where(valid, probs, -2.0)
    val0 = jnp.where(real, -1.0, -2.0)

    def nms_cond(carry):
        t, _, _, g = carry
        return (t < DET) & (g > SCORE_THRESH)

    def nms_body(carry):
        t, work, val, _ = carry
        m = jnp.max(work, axis=0, keepdims=True)
        lane_active = m > SCORE_THRESH
        cand = work == m
        first = jnp.min(jnp.where(cand, row_idx, BIG), axis=0, keepdims=True)
        b = cand & (row_idx == first) & lane_active
        xi1 = jnp.sum(jnp.where(b, x1, 0.0), axis=0, keepdims=True)
        yi1 = jnp.sum(jnp.where(b, y1, 0.0), axis=0, keepdims=True)
        xi2 = jnp.sum(jnp.where(b, x2, 0.0), axis=0, keepdims=True)
        yi2 = jnp.sum(jnp.where(b, y2, 0.0), axis=0, keepdims=True)
        ai = (xi2 - xi1 + 1.0) * (yi2 - yi1 + 1.0)
        xx1 = jnp.maximum(x1, xi1)
        yy1 = jnp.maximum(y1, yi1)
        xx2 = jnp.minimum(x2, xi2)
        yy2 = jnp.minimum(y2, yi2)
        w = jnp.maximum(xx2 - xx1 + 1.0, 0.0)
        h = jnp.maximum(yy2 - yy1 + 1.0, 0.0)
        inter = w * h
        iou = inter / (area + ai - inter)
        val = jnp.where(b, m, val)
        work = jnp.where(lane_active & (iou > NMS_THRESH), -2.0, work)
        return t + 1, work, val, jnp.max(work)

    g0 = jnp.max(work0)
    _, _, val, _ = jax.lax.while_loop(nms_cond, nms_body, (0, work0, val0, g0))

    sx1[...] = x1
    sy1[...] = y1
    sx2[...] = x2
    sy2[...] = y2
    det_ref[...] = jnp.zeros((DET_PAD, LANES), jnp.float32)

    def sel_body(t, carry):
        v, labv = carry
        pm = jnp.max(v, axis=0, keepdims=True)
        m = jnp.max(pm)
        lane_min = jnp.min(jnp.where(pm == m, lanes, BIG))
        candfull = (v == m) & (lanes == lane_min)
        rmin = jnp.min(jnp.where(candfull, row_idx, BIG))
        winner = candfull & (row_idx == rmin)
        lmask = lanes == lane_min
        bx1 = jnp.sum(jnp.where(lmask, sx1[pl.ds(rmin, 1), :], 0.0))
        by1 = jnp.sum(jnp.where(lmask, sy1[pl.ds(rmin, 1), :], 0.0))
        bx2 = jnp.sum(jnp.where(lmask, sx2[pl.ds(rmin, 1), :], 0.0))
        by2 = jnp.sum(jnp.where(lmask, sy2[pl.ds(rmin, 1), :], 0.0))
        rowv = jnp.where(lanes == 0, bx1,
               jnp.where(lanes == 1, by1,
               jnp.where(lanes == 2, bx2,
               jnp.where(lanes == 3, by2,
               jnp.where(lanes == 4, m, 0.0)))))
        det_ref[pl.ds(t, 1), :] = rowv
        labv = jnp.where(lanes == t, lane_min, labv)
        v = jnp.where(winner, -3.0, v)
        return v, labv

    _, labv = jax.lax.fori_loop(
        0, DET, sel_body, (val, jnp.zeros((1, LANES), jnp.int32)))
    lab_ref[...] = jnp.broadcast_to(labv, (8, LANES))


def kernel(class_logits, box_regression, proposals):
    lg = jnp.pad(class_logits, ((0, NP - N), (0, 0)))
    lg = jnp.pad(lg, ((0, 0), (0, LANES - C)), constant_values=NEG)
    r4 = box_regression.reshape(N, C, 4)

    def padc(a):
        return jnp.pad(a, ((0, NP - N), (0, LANES - C)))

    dxi = padc(r4[:, :, 0])
    dyi = padc(r4[:, :, 1])
    dwi = padc(r4[:, :, 2])
    dhi = padc(r4[:, :, 3])
    props = jnp.pad(proposals, ((0, NP - N), (0, 0)))
    det_full, lab_full = pl.pallas_call(
        _body,
        out_shape=[jax.ShapeDtypeStruct((DET_PAD, LANES), jnp.float32),
                   jax.ShapeDtypeStruct((8, LANES), jnp.int32)],
        scratch_shapes=[pltpu.VMEM((NP, LANES), jnp.float32)] * 4,
        interpret=_INTERPRET,
    )(lg, dxi, dyi, dwi, dhi, props)
    det = det_full[:DET, :5]
    labels = lab_full[0, :DET]
    return det, labels
